# baseline (speedup 1.0000x reference)
import functools

import jax
import jax.numpy as jnp
import numpy as np
from jax import lax
from jax.experimental import pallas as pl
from jax.experimental.pallas import tpu as pltpu
from jax.experimental.pallas import tpu_sc as plsc

_MAGIC = 12582912.0
_PI2_HI = float(np.float32(2.0 * np.pi))
_PI2_LO = float(np.float32(2.0 * np.pi - np.float64(np.float32(2.0 * np.pi))))
_SIN_C = [0.9999999403953552, -0.1666662096977234, 0.008332791738212109,
          -0.00019817630527541041, 2.708831061681849e-06,
          -2.069813476168747e-08]
_COS_C = [1.0, -0.49999988079071045, 0.04166648909449577,
          -0.0013887803070247173, 2.4769884475972503e-05,
          -2.707903092868946e-07, 1.7245092021056507e-09]

_B, _F, _P = 16384, 512, 128
_NW = 32
_ROWS_W = _B // _NW
_R = 64
_NCHUNK = _ROWS_W // _R


def _sc_call(sc_mat, x):
    mesh = plsc.VectorSubcoreMesh(core_axis_name="c", subcore_axis_name="s")

    @functools.partial(
        pl.kernel, mesh=mesh,
        out_type=jax.ShapeDtypeStruct((_B, _F + _P), jnp.float32),
        compiler_params=pltpu.CompilerParams(use_tc_tiling_on_sc=False,
                                             needs_layout_passes=False),
        scratch_types=[
            pltpu.VMEM((4, 16), jnp.float32),
            pltpu.VMEM((_R, _P), jnp.float32),
            pltpu.VMEM((_R, 2 * _P), jnp.float32),
        ],
    )
    def k(sc_hbm, x_hbm, out_hbm, sc_v, xp_v, outp_v):
        wid = lax.axis_index("s") * 2 + lax.axis_index("c")
        base0 = wid * _ROWS_W
        pltpu.sync_copy(sc_hbm, sc_v)
        scale = sc_v[0]
        red_a = sc_v[1]
        red_b = sc_v[2]
        aff_c = sc_v[3]
        iota2 = 2 * lax.broadcasted_iota(jnp.int32, (16,), 0)

        def chunk_body(ci, carry):
            base = base0 + ci * _R
            pltpu.sync_copy(x_hbm.at[pl.ds(base, _R), pl.ds(0, _P)], xp_v)
            pltpu.sync_copy(x_hbm.at[pl.ds(base, _R), pl.ds(_P, _F - _P)],
                            out_hbm.at[pl.ds(base, _R), pl.ds(2 * _P, _F - _P)])

            def row_body(rr, c2):
                orow = outp_v.at[rr]
                for j in range(8):
                    v = xp_v[rr, pl.ds(16 * j, 16)]
                    kf = v * red_a + red_b
                    kk = (kf + _MAGIC) - _MAGIC
                    rad = (v * scale + aff_c) - kk * _PI2_HI - kk * _PI2_LO
                    z = rad * rad
                    cacc = jnp.full((16,), _COS_C[6], jnp.float32)
                    for t in range(5, -1, -1):
                        cacc = cacc * z + _COS_C[t]
                    sacc = jnp.full((16,), _SIN_C[5], jnp.float32)
                    for t in range(4, -1, -1):
                        sacc = sacc * z + _SIN_C[t]
                    sacc = sacc * rad
                    plsc.store_scatter(orow, [32 * j + iota2], cacc)
                    plsc.store_scatter(orow, [32 * j + 1 + iota2], sacc)
                return c2

            lax.fori_loop(0, _R, row_body, 0)
            pltpu.sync_copy(outp_v, out_hbm.at[pl.ds(base, _R), pl.ds(0, 2 * _P)])
            return carry

        lax.fori_loop(0, _NCHUNK, chunk_body, 0)

    return k(sc_mat, x)


def kernel(x, limits, periodic_indices, nonperiodic_indices,
           periodic_indices_lifted, nonperiodic_indices_lifted):
    scale = 2.0 * jnp.pi / (limits[1] - limits[0])
    shift = limits[0]
    inv2pi = 1.0 / (2.0 * np.pi)
    sc4 = jnp.stack([scale, scale * inv2pi, -shift * scale * inv2pi,
                     -shift * scale]).astype(jnp.float32)
    sc_mat = jnp.broadcast_to(sc4[:, None], (4, 16))
    return _sc_call(sc_mat, x)


# speedup vs baseline: 1.0140x; 1.0140x over previous
import functools

import jax
import jax.numpy as jnp
import numpy as np
from jax import lax
from jax.experimental import pallas as pl
from jax.experimental.pallas import tpu as pltpu
from jax.experimental.pallas import tpu_sc as plsc

_MAGIC = 12582912.0
_PI2_HI = float(np.float32(2.0 * np.pi))
_PI2_LO = float(np.float32(2.0 * np.pi - np.float64(np.float32(2.0 * np.pi))))
_SIN_C = [0.9999999403953552, -0.1666662096977234, 0.008332791738212109,
          -0.00019817630527541041, 2.708831061681849e-06,
          -2.069813476168747e-08]
_COS_C = [1.0, -0.49999988079071045, 0.04166648909449577,
          -0.0013887803070247173, 2.4769884475972503e-05,
          -2.707903092868946e-07, 1.7245092021056507e-09]

_B, _F, _P = 16384, 512, 128
_NW = 32
_ROWS_W = _B // _NW
_R = 64
_NCHUNK = _ROWS_W // _R
_UNROLL = 8


def _sc_call(sc_mat, x):
    mesh = plsc.VectorSubcoreMesh(core_axis_name="c", subcore_axis_name="s")

    @functools.partial(
        pl.kernel, mesh=mesh,
        out_type=jax.ShapeDtypeStruct((_B, _F + _P), jnp.float32),
        compiler_params=pltpu.CompilerParams(use_tc_tiling_on_sc=False,
                                             needs_layout_passes=False),
        scratch_types=[
            pltpu.VMEM((4, 16), jnp.float32),
            pltpu.VMEM((2, _R, _P), jnp.float32),
            pltpu.VMEM((2, _R, 2 * _P), jnp.float32),
            pltpu.SemaphoreType.DMA,
            pltpu.SemaphoreType.DMA,
            pltpu.SemaphoreType.DMA,
        ],
    )
    def k(sc_hbm, x_hbm, out_hbm, sc_v, xp_v, outp_v, sem_in, sem_out, sem_cp):
        wid = lax.axis_index("s") * 2 + lax.axis_index("c")
        base0 = wid * _ROWS_W
        pltpu.sync_copy(sc_hbm, sc_v)
        scale = sc_v[0]
        red_a = sc_v[1]
        red_b = sc_v[2]
        aff_c = sc_v[3]
        iota2 = 2 * lax.broadcasted_iota(jnp.int32, (16,), 0)

        cp_descs = []
        for ci in range(_NCHUNK):
            base = base0 + ci * _R
            cp_descs.append(pltpu.make_async_copy(
                x_hbm.at[pl.ds(base, _R), pl.ds(_P, _F - _P)],
                out_hbm.at[pl.ds(base, _R), pl.ds(2 * _P, _F - _P)],
                sem_cp))
        for d in cp_descs:
            d.start()

        def in_desc(ci, buf):
            base = base0 + ci * _R
            return pltpu.make_async_copy(
                x_hbm.at[pl.ds(base, _R), pl.ds(0, _P)], xp_v.at[buf], sem_in)

        def out_desc(ci, buf):
            base = base0 + ci * _R
            return pltpu.make_async_copy(
                outp_v.at[buf], out_hbm.at[pl.ds(base, _R), pl.ds(0, 2 * _P)],
                sem_out)

        def compute(buf):
            xp = xp_v.at[buf]
            op = outp_v.at[buf]

            @plsc.parallel_loop(0, _R * 8, unroll=_UNROLL)
            def body(i):
                rr = lax.shift_right_logical(i, 3)
                jj = lax.bitwise_and(i, 7)
                col = 32 * jj
                v = xp[rr, pl.ds(16 * jj, 16)]
                kf = v * red_a + red_b
                kk = (kf + _MAGIC) - _MAGIC
                rad = (v * scale + aff_c) - kk * _PI2_HI - kk * _PI2_LO
                z = rad * rad
                cacc = jnp.full((16,), _COS_C[6], jnp.float32)
                for t in range(5, -1, -1):
                    cacc = cacc * z + _COS_C[t]
                sacc = jnp.full((16,), _SIN_C[5], jnp.float32)
                for t in range(4, -1, -1):
                    sacc = sacc * z + _SIN_C[t]
                sacc = sacc * rad
                orow = op.at[rr]
                plsc.store_scatter(orow, [col + iota2], cacc)
                plsc.store_scatter(orow, [col + 1 + iota2], sacc)

        in_descs = [in_desc(ci, ci % 2) for ci in range(_NCHUNK)]
        out_descs = [out_desc(ci, ci % 2) for ci in range(_NCHUNK)]
        in_descs[0].start()
        for ci in range(_NCHUNK):
            buf = ci % 2
            if ci + 1 < _NCHUNK:
                in_descs[ci + 1].start()
            in_descs[ci].wait()
            if ci >= 2:
                out_descs[ci - 2].wait()
            compute(buf)
            out_descs[ci].start()
        out_descs[_NCHUNK - 2].wait()
        out_descs[_NCHUNK - 1].wait()
        for d in cp_descs:
            d.wait()

    return k(sc_mat, x)


def kernel(x, limits, periodic_indices, nonperiodic_indices,
           periodic_indices_lifted, nonperiodic_indices_lifted):
    scale = 2.0 * jnp.pi / (limits[1] - limits[0])
    shift = limits[0]
    inv2pi = 1.0 / (2.0 * np.pi)
    sc4 = jnp.stack([scale, scale * inv2pi, -shift * scale * inv2pi,
                     -shift * scale]).astype(jnp.float32)
    sc_mat = jnp.broadcast_to(sc4[:, None], (4, 16))
    return _sc_call(sc_mat, x)


# speedup vs baseline: 6.7489x; 6.6557x over previous
import functools

import jax
import jax.numpy as jnp
import numpy as np
from jax import lax
from jax.experimental import pallas as pl
from jax.experimental.pallas import tpu as pltpu
from jax.experimental.pallas import tpu_sc as plsc

_MAGIC = 12582912.0
_PI2_HI = float(np.float32(2.0 * np.pi))
_PI2_LO = float(np.float32(2.0 * np.pi - np.float64(np.float32(2.0 * np.pi))))
_SIN_C = [0.9999999403953552, -0.1666662096977234, 0.008332791738212109,
          -0.00019817630527541041, 2.708831061681849e-06,
          -2.069813476168747e-08]
_COS_C = [1.0, -0.49999988079071045, 0.04166648909449577,
          -0.0013887803070247173, 2.4769884475972503e-05,
          -2.707903092868946e-07, 1.7245092021056507e-09]

_B, _F, _P = 16384, 512, 128
_NW = 32
_ROWS_W = _B // _NW
_R = 32
_NCHUNK = _ROWS_W // _R
_UNROLL = 8


def _sc_call(sc_mat, x):
    mesh = plsc.VectorSubcoreMesh(core_axis_name="c", subcore_axis_name="s")

    @functools.partial(
        pl.kernel, mesh=mesh,
        out_type=jax.ShapeDtypeStruct((_B, _F + _P), jnp.float32),
        compiler_params=pltpu.CompilerParams(use_tc_tiling_on_sc=False,
                                             needs_layout_passes=False),
        scratch_types=[
            pltpu.VMEM((4, 16), jnp.float32),
            pltpu.VMEM((2, _R, _F), jnp.float32),
            pltpu.VMEM((2, _R, _F + _P), jnp.float32),
            pltpu.SemaphoreType.DMA,
            pltpu.SemaphoreType.DMA,
        ],
    )
    def k(sc_hbm, x_hbm, out_hbm, sc_v, xin_v, outp_v, sem_in, sem_out):
        wid = lax.axis_index("s") * 2 + lax.axis_index("c")
        base0 = wid * _ROWS_W
        pltpu.sync_copy(sc_hbm, sc_v)
        scale = sc_v[0]
        red_a = sc_v[1]
        red_b = sc_v[2]
        aff_c = sc_v[3]
        iota2 = 2 * lax.broadcasted_iota(jnp.int32, (16,), 0)

        def in_start(ci, buf):
            base = base0 + ci * _R
            return pltpu.make_async_copy(
                x_hbm.at[pl.ds(base, _R)], xin_v.at[buf], sem_in).start()

        def in_wait(buf):
            pltpu.make_async_copy(
                x_hbm.at[pl.ds(0, _R)], xin_v.at[buf], sem_in).wait()

        def out_start(ci, buf):
            base = base0 + ci * _R
            return pltpu.make_async_copy(
                outp_v.at[buf], out_hbm.at[pl.ds(base, _R)], sem_out).start()

        def out_wait(buf):
            pltpu.make_async_copy(
                outp_v.at[buf], out_hbm.at[pl.ds(0, _R)], sem_out).wait()

        def compute(buf):
            xin = xin_v.at[buf]
            op = outp_v.at[buf]

            @plsc.parallel_loop(0, _R * 8, unroll=_UNROLL)
            def body(i):
                rr = lax.shift_right_logical(i, 3)
                jj = lax.bitwise_and(i, 7)
                col = 32 * jj
                v = xin[rr, pl.ds(16 * jj, 16)]
                kf = v * red_a + red_b
                kk = (kf + _MAGIC) - _MAGIC
                rad = (v * scale + aff_c) - kk * _PI2_HI - kk * _PI2_LO
                z = rad * rad
                cacc = jnp.full((16,), _COS_C[6], jnp.float32)
                for t in range(5, -1, -1):
                    cacc = cacc * z + _COS_C[t]
                sacc = jnp.full((16,), _SIN_C[5], jnp.float32)
                for t in range(4, -1, -1):
                    sacc = sacc * z + _SIN_C[t]
                sacc = sacc * rad
                orow = op.at[rr]
                plsc.store_scatter(orow, [col + iota2], cacc)
                plsc.store_scatter(orow, [col + 1 + iota2], sacc)

            @plsc.parallel_loop(0, _R * 24, unroll=_UNROLL)
            def copy_body(i):
                rr = i // 24
                jj = i - rr * 24
                src = 128 + 16 * jj
                op[rr, pl.ds(src + 128, 16)] = xin[rr, pl.ds(src, 16)]

        in_start(0, 0)
        in_start(1, 1)

        @pl.loop(0, _NCHUNK, step=2)
        def chunks(ci0):
            for b in range(2):
                ci = ci0 + b
                in_wait(b)

                @pl.when(ci >= 2)
                def _():
                    out_wait(b)

                compute(b)
                out_start(ci, b)

                @pl.when(ci + 2 < _NCHUNK)
                def _():
                    in_start(ci + 2, b)

        out_wait(0)
        out_wait(1)

    return k(sc_mat, x)


def kernel(x, limits, periodic_indices, nonperiodic_indices,
           periodic_indices_lifted, nonperiodic_indices_lifted):
    scale = 2.0 * jnp.pi / (limits[1] - limits[0])
    shift = limits[0]
    inv2pi = 1.0 / (2.0 * np.pi)
    sc4 = jnp.stack([scale, scale * inv2pi, -shift * scale * inv2pi,
                     -shift * scale]).astype(jnp.float32)
    sc_mat = jnp.broadcast_to(sc4[:, None], (4, 16))
    return _sc_call(sc_mat, x)
